# baseline (speedup 1.0000x reference)
import functools

import jax
import jax.numpy as jnp
from jax import lax
from jax.experimental import pallas as pl
from jax.experimental.pallas import tpu as pltpu
from jax.experimental.pallas import tpu_sc as plsc

N_TOKENS = 131072
EMBED_DIM = 32
N_LATENTS = 1024
BLOCK = 4096

_NUM_WORKERS = 32
_ROWS_PER_WORKER = N_TOKENS // _NUM_WORKERS
_CHUNK = 1024


def _vq_block_kernel(xt_ref, cbm2_ref, csq_ref, zrows_ref, z_ref, loss_ref):
    i = pl.program_id(0)
    xt = xt_ref[...]
    cbm2 = cbm2_ref[...]

    scorest = jax.lax.dot_general(
        cbm2, xt, (((1,), (0,)), ((), ())),
        preferred_element_type=jnp.float32,
    )
    adjt = scorest + csq_ref[...]

    amin = jnp.min(adjt, axis=0, keepdims=True)
    ind = (adjt == amin).astype(jnp.bfloat16)

    zt = jax.lax.dot_general(
        zrows_ref[...], ind, (((1,), (0,)), ((), ())),
        preferred_element_type=jnp.float32,
    )
    z = jnp.minimum(zt[0, :] + zt[1, :], float(N_LATENTS - 1))
    z_ref[...] = z.astype(jnp.int32)

    x_sq = jnp.sum(xt * xt, axis=0)
    part = (jnp.sum(amin) + jnp.sum(x_sq))[None, None]

    @pl.when(i == 0)
    def _():
        loss_ref[...] = jnp.zeros_like(loss_ref)

    loss_ref[...] += part


@functools.lru_cache(maxsize=None)
def _make_sc_gather(n_rows):
    rows_per_worker = n_rows // _NUM_WORKERS

    @functools.partial(
        pl.kernel,
        mesh=plsc.VectorSubcoreMesh(core_axis_name="c", subcore_axis_name="s"),
        out_type=jax.ShapeDtypeStruct((n_rows * EMBED_DIM,), jnp.float32),
        compiler_params=pltpu.CompilerParams(needs_layout_passes=False),
        scratch_types=[
            pltpu.VMEM((N_LATENTS * EMBED_DIM,), jnp.float32),
            pltpu.VMEM((_CHUNK,), jnp.int32),
            pltpu.VMEM((_CHUNK * EMBED_DIM,), jnp.float32),
        ],
    )
    def _sc_gather(cb_hbm, z_hbm, out_hbm, cb_v, idx_v, rows_v):
        wid = lax.axis_index("s") * 2 + lax.axis_index("c")
        pltpu.sync_copy(cb_hbm, cb_v)

        for c in range(rows_per_worker // _CHUNK):
            base = wid * rows_per_worker + c * _CHUNK
            pltpu.sync_copy(z_hbm.at[pl.ds(base, _CHUNK)], idx_v)

            @plsc.parallel_loop(0, _CHUNK // 16, 1, unroll=2)
            def _(g):
                zv = idx_v[pl.ds(g * 16, 16)] * EMBED_DIM
                for j in range(16):
                    src = zv[j]
                    dst = (g * 16 + j) * EMBED_DIM
                    rows_v[pl.ds(dst, 16)] = cb_v[pl.ds(src, 16)]
                    rows_v[pl.ds(dst + 16, 16)] = cb_v[pl.ds(src + 16, 16)]

            pltpu.sync_copy(
                rows_v,
                out_hbm.at[pl.ds(base * EMBED_DIM, _CHUNK * EMBED_DIM)])

    return _sc_gather


def _make_zrows():
    k = jnp.arange(N_LATENTS, dtype=jnp.int32)
    hi = ((k // 8) * 8).astype(jnp.float32)
    lo = (k % 8).astype(jnp.float32)
    rows = jnp.stack([hi, lo] + [jnp.zeros(N_LATENTS)] * 6, axis=0)
    return rows.astype(jnp.bfloat16)


def _vq_argmin_half(xt_half, cbm2, csq, zrows, nh):
    grid = nh // BLOCK
    return pl.pallas_call(
        _vq_block_kernel,
        grid=(grid,),
        in_specs=[
            pl.BlockSpec((EMBED_DIM, BLOCK), lambda i: (0, i)),
            pl.BlockSpec((N_LATENTS, EMBED_DIM), lambda i: (0, 0)),
            pl.BlockSpec((N_LATENTS, 1), lambda i: (0, 0)),
            pl.BlockSpec((8, N_LATENTS), lambda i: (0, 0)),
        ],
        out_specs=[
            pl.BlockSpec((BLOCK,), lambda i: (i,)),
            pl.BlockSpec((1, 1), lambda i: (0, 0)),
        ],
        out_shape=[
            jax.ShapeDtypeStruct((nh,), jnp.int32),
            jax.ShapeDtypeStruct((1, 1), jnp.float32),
        ],
    )(xt_half, cbm2, csq, zrows)


@jax.jit
def kernel(x, codebook):
    n, d = x.shape
    nh = n // 2
    xt = x.T
    cbm2 = -2.0 * codebook
    csq = jnp.sum(codebook * codebook, axis=1, keepdims=True)
    zrows = _make_zrows()
    cb_flat = codebook.reshape(-1)

    sc_gather = _make_sc_gather(nh)
    z1, loss1 = _vq_argmin_half(xt[:, :nh], cbm2, csq, zrows, nh)
    q1 = sc_gather(cb_flat, z1)
    z2, loss2 = _vq_argmin_half(xt[:, nh:], cbm2, csq, zrows, nh)
    q2 = sc_gather(cb_flat, z2)

    z = jnp.concatenate([z1, z2])
    q = jnp.concatenate([q1, q2]).reshape(n, d)

    loss = (loss1[0, 0] + loss2[0, 0]) / (n * d)
    return (z, q, (loss, loss))


# speedup vs baseline: 1.0036x; 1.0036x over previous
import functools

import jax
import jax.numpy as jnp
from jax import lax
from jax.experimental import pallas as pl
from jax.experimental.pallas import tpu as pltpu
from jax.experimental.pallas import tpu_sc as plsc

N_TOKENS = 131072
EMBED_DIM = 32
N_LATENTS = 1024
BLOCK = 4096

_NUM_WORKERS = 32
_ROWS_PER_WORKER = N_TOKENS // _NUM_WORKERS
_CHUNK = 1024


def _vq_block_kernel(xt_ref, cbm2_ref, csq_ref, zrows_ref, z_ref, loss_ref):
    i = pl.program_id(0)
    xt = xt_ref[...]
    cbm2 = cbm2_ref[...]

    scorest = jax.lax.dot_general(
        cbm2, xt, (((1,), (0,)), ((), ())),
        preferred_element_type=jnp.float32,
    )
    adjt = scorest + csq_ref[...]

    amin = jnp.min(adjt, axis=0, keepdims=True)
    ind = (adjt == amin).astype(jnp.bfloat16)

    zt = jax.lax.dot_general(
        zrows_ref[...], ind, (((1,), (0,)), ((), ())),
        preferred_element_type=jnp.float32,
    )
    z = jnp.minimum(zt[0, :] + zt[1, :], float(N_LATENTS - 1))
    z_ref[...] = z.astype(jnp.int32)

    x_sq = jnp.sum(xt * xt, axis=0)
    part = (jnp.sum(amin) + jnp.sum(x_sq))[None, None]

    @pl.when(i == 0)
    def _():
        loss_ref[...] = jnp.zeros_like(loss_ref)

    loss_ref[...] += part


@functools.lru_cache(maxsize=None)
def _make_sc_gather(n_rows):
    rows_per_worker = n_rows // _NUM_WORKERS

    @functools.partial(
        pl.kernel,
        mesh=plsc.VectorSubcoreMesh(core_axis_name="c", subcore_axis_name="s"),
        out_type=jax.ShapeDtypeStruct((n_rows * EMBED_DIM,), jnp.float32),
        compiler_params=pltpu.CompilerParams(needs_layout_passes=False),
        scratch_types=[
            pltpu.VMEM((N_LATENTS * EMBED_DIM,), jnp.float32),
            pltpu.VMEM((_CHUNK,), jnp.int32),
            pltpu.VMEM((_CHUNK * EMBED_DIM,), jnp.float32),
        ],
    )
    def _sc_gather(cb_hbm, z_hbm, out_hbm, cb_v, idx_v, rows_v):
        wid = lax.axis_index("s") * 2 + lax.axis_index("c")
        pltpu.sync_copy(cb_hbm, cb_v)

        for c in range(rows_per_worker // _CHUNK):
            base = wid * rows_per_worker + c * _CHUNK
            pltpu.sync_copy(z_hbm.at[pl.ds(base, _CHUNK)], idx_v)

            @plsc.parallel_loop(0, _CHUNK // 16, 1, unroll=2)
            def _(g):
                zv = idx_v[pl.ds(g * 16, 16)] * EMBED_DIM
                for j in range(16):
                    src = zv[j]
                    dst = (g * 16 + j) * EMBED_DIM
                    rows_v[pl.ds(dst, 16)] = cb_v[pl.ds(src, 16)]
                    rows_v[pl.ds(dst + 16, 16)] = cb_v[pl.ds(src + 16, 16)]

            pltpu.sync_copy(
                rows_v,
                out_hbm.at[pl.ds(base * EMBED_DIM, _CHUNK * EMBED_DIM)])

    return _sc_gather


def _make_zrows():
    k = jnp.arange(N_LATENTS, dtype=jnp.int32)
    hi = ((k // 8) * 8).astype(jnp.float32)
    lo = (k % 8).astype(jnp.float32)
    rows = jnp.stack([hi, lo] + [jnp.zeros(N_LATENTS)] * 6, axis=0)
    return rows.astype(jnp.bfloat16)


def _vq_argmin_half(xt_half, cbm2, csq, zrows, nh):
    grid = nh // BLOCK
    return pl.pallas_call(
        _vq_block_kernel,
        grid=(grid,),
        in_specs=[
            pl.BlockSpec((EMBED_DIM, BLOCK), lambda i: (0, i)),
            pl.BlockSpec((N_LATENTS, EMBED_DIM), lambda i: (0, 0)),
            pl.BlockSpec((N_LATENTS, 1), lambda i: (0, 0)),
            pl.BlockSpec((8, N_LATENTS), lambda i: (0, 0)),
        ],
        out_specs=[
            pl.BlockSpec((BLOCK,), lambda i: (i,)),
            pl.BlockSpec((1, 1), lambda i: (0, 0)),
        ],
        out_shape=[
            jax.ShapeDtypeStruct((nh,), jnp.int32),
            jax.ShapeDtypeStruct((1, 1), jnp.float32),
        ],
    )(xt_half, cbm2, csq, zrows)


@jax.jit
def kernel(x, codebook):
    n, d = x.shape
    xt = x.T
    cbm2 = -2.0 * codebook
    csq = jnp.sum(codebook * codebook, axis=1, keepdims=True)
    zrows = _make_zrows()

    z, loss_sum = _vq_argmin_half(xt, cbm2, csq, zrows, n)
    q = _make_sc_gather(n)(codebook.reshape(-1), z).reshape(n, d)

    loss = loss_sum[0, 0] / (n * d)
    return (z, q, (loss, loss))
